# Initial kernel scaffold; baseline (speedup 1.0000x reference)
import jax
import jax.numpy as jnp
from jax.experimental import pallas as pl


def kernel(quantized, neighbor_idx):
    raise NotImplementedError("write your pallas kernel here")


# speedup vs baseline: 38.6369x; 38.6369x over previous
import jax
import jax.numpy as jnp
from jax import lax
from jax.experimental import pallas as pl
from jax.experimental.pallas import tpu as pltpu
from jax.experimental.pallas import tpu_sc as plsc

_NC = 2
_NS = 16
_NW = _NC * _NS
_LANES = 16

_B, _T, _D = 16, 2048, 512
_ROWS = _B * _T
_RPW = _ROWS // _NW
_CHUNK = 64
_NCHUNK = _RPW // _CHUNK


def _jitter_body(q_hbm, nb_hbm, out_hbm, nb_v, idx_v, buf_v, sem):
    wid = lax.axis_index("s") * _NC + lax.axis_index("c")
    base = wid * _RPW

    pltpu.sync_copy(nb_hbm, nb_v)

    iota = lax.iota(jnp.int32, _LANES)

    def fill_ident(c, _):
        idx_v[pl.ds(c * _LANES, _LANES)] = base + c * _LANES + iota
        return 0

    lax.fori_loop(0, _RPW // _LANES, fill_ident, 0)

    @pl.when(wid % (_T // _RPW) == 0)
    def _():
        def resolve(m, carry):
            t0 = m * _LANES
            nbv = nb_v[pl.ds(t0, _LANES)]
            tv = t0 + iota
            h = jnp.where(nbv == tv - 1, -1, nbv)
            v = jnp.maximum(plsc.cummax(h), carry)
            idx_v[pl.ds(t0, _LANES)] = base + v
            return jnp.max(v)

        lax.fori_loop(0, _D // _LANES, resolve, jnp.int32(-1))

    for k in range(_NCHUNK):
        idx_slice = idx_v.at[pl.ds(k * _CHUNK, _CHUNK)]
        pltpu.async_copy(q_hbm.at[idx_slice], buf_v, sem).wait()
        pltpu.sync_copy(buf_v, out_hbm.at[pl.ds(base + k * _CHUNK, _CHUNK)])


def kernel(quantized, neighbor_idx):
    q2d = quantized.reshape(_ROWS, _D)
    nb = jnp.asarray(neighbor_idx, jnp.int32)

    mesh = plsc.VectorSubcoreMesh(core_axis_name="c", subcore_axis_name="s")
    out = pl.kernel(
        _jitter_body,
        out_type=jax.ShapeDtypeStruct((_ROWS, _D), jnp.float32),
        mesh=mesh,
        scratch_types=[
            pltpu.VMEM((_D,), jnp.int32),
            pltpu.VMEM((_RPW,), jnp.int32),
            pltpu.VMEM((_CHUNK, _D), jnp.float32),
            pltpu.SemaphoreType.DMA,
        ],
        compiler_params=pltpu.CompilerParams(needs_layout_passes=False),
    )(q2d, nb)
    return out.reshape(_B, _T, _D)
